# Initial kernel scaffold; baseline (speedup 1.0000x reference)
import functools

import jax
import jax.numpy as jnp
from jax import lax
from jax.experimental import pallas as pl
from jax.experimental.pallas import tpu as pltpu
from jax.experimental.pallas import tpu_sc as plsc

N = 10000
F = 128
NW = 32
CHUNK = 320
NP = NW * CHUNK
PAD = NP - N
NEG = -1e30
NIDX = 80
NCH = CHUNK // NIDX
L = 16
TILE = 128
NT = NP // TILE
T2 = 512
NT2 = NP // T2
SEARCH_STEPS = 14


def _proj_body(x_ref, w_ref, asrc_ref, adst_ref, h_ref, s_ref, d_ref):
    h = jnp.dot(x_ref[...], w_ref[...], preferred_element_type=jnp.float32)
    h_ref[...] = h
    s_ref[...] = jnp.sum(h * asrc_ref[...], axis=1, keepdims=True)
    d_ref[...] = jnp.sum(h * adst_ref[...], axis=1, keepdims=True)


_proj = pl.pallas_call(
    _proj_body,
    out_shape=[
        jax.ShapeDtypeStruct((N, F), jnp.float32),
        jax.ShapeDtypeStruct((N, 1), jnp.float32),
        jax.ShapeDtypeStruct((N, 1), jnp.float32),
    ],
)


_mesh = plsc.VectorSubcoreMesh(core_axis_name="c", subcore_axis_name="s")


@functools.partial(
    pl.kernel,
    mesh=_mesh,
    out_type=[
        jax.ShapeDtypeStruct((NP, F), jnp.float32),
        jax.ShapeDtypeStruct((NP,), jnp.int32),
    ],
    scratch_types=[
        pltpu.VMEM((NP,), jnp.float32),
        pltpu.VMEM((CHUNK,), jnp.float32),
        pltpu.VMEM((NIDX,), jnp.int32),
        pltpu.VMEM((NIDX,), jnp.int32),
        pltpu.VMEM((NIDX,), jnp.int32),
        pltpu.VMEM((NIDX,), jnp.int32),
        pltpu.VMEM((CHUNK,), jnp.int32),
        pltpu.VMEM((CHUNK, F), jnp.float32),
        pltpu.SemaphoreType.DMA,
    ],
)
def _sc_rank(p_hbm, s_hbm, ds_hbm, h_hbm, hp_hbm, g_hbm,
             ds_v, s_v, p0, p1, p2, p3, g_v, rows_v, sem):
    wid = lax.axis_index("s") * 2 + lax.axis_index("c")
    base = wid * CHUNK
    pbufs = (p0, p1, p2, p3)
    for c in range(NCH):
        pltpu.sync_copy(p_hbm.at[pl.ds(base + c * NIDX, NIDX)], pbufs[c])
    cps = []
    for c in range(NCH):
        cps.append(pltpu.async_copy(
            h_hbm.at[pbufs[c]], rows_v.at[pl.ds(c * NIDX, NIDX)], sem))
    pltpu.sync_copy(ds_hbm, ds_v)
    pltpu.sync_copy(s_hbm.at[pl.ds(base, CHUNK)], s_v)

    def search_one(v, carry):
        t = -s_v[pl.ds(v * L, L)]
        lo = jnp.zeros((L,), jnp.int32)
        hi = jnp.full((L,), NP, jnp.int32)
        for _ in range(SEARCH_STEPS):
            mid = (lo + hi) >> 1
            dm = plsc.load_gather(ds_v, [mid])
            go = dm <= t
            lo = jnp.where(go, mid + 1, lo)
            hi = jnp.where(go, hi, mid)
        g_v[pl.ds(v * L, L)] = jnp.maximum(lo - 1, 0)
        return carry

    lax.fori_loop(0, CHUNK // L, search_one, 0)
    pltpu.sync_copy(g_v, g_hbm.at[pl.ds(base, CHUNK)])
    for cp in cps:
        cp.wait()
    pltpu.sync_copy(rows_v, hp_hbm.at[pl.ds(base, CHUNK)])


def _cumsum_body(dmax_ref, ds_ref, hp_ref, c1_ref, c2_ref, z1_ref, z2_ref,
                 tot_ref, c1c, c2c, z1c, z2c):
    i = pl.program_id(0)

    @pl.when(i == 0)
    def _init():
        c1c[...] = jnp.zeros_like(c1c)
        c2c[...] = jnp.zeros_like(c2c)
        z1c[...] = jnp.zeros_like(z1c)
        z2c[...] = jnp.zeros_like(z2c)

    rid = lax.broadcasted_iota(jnp.int32, (TILE, TILE), 0)
    cid = lax.broadcasted_iota(jnp.int32, (TILE, TILE), 1)
    lt = jnp.where(rid >= cid, 1.0, 0.0).astype(jnp.float32)
    dcol = ds_ref[...]
    dmax = dmax_ref[...]
    w1 = jnp.exp(dcol - dmax)
    w2 = jnp.exp(0.2 * (dcol - dmax))
    tile = hp_ref[...]
    cs1 = jnp.dot(lt, w1 * tile, preferred_element_type=jnp.float32) + c1c[...]
    cs2 = jnp.dot(lt, w2 * tile, preferred_element_type=jnp.float32) + c2c[...]
    zc1 = jnp.dot(lt, w1, preferred_element_type=jnp.float32) + z1c[...]
    zc2 = jnp.dot(lt, w2, preferred_element_type=jnp.float32) + z2c[...]
    c1_ref[...] = cs1
    c2_ref[...] = cs2
    z1_ref[...] = zc1
    z2_ref[...] = zc2
    c1c[...] = cs1[TILE - 1:TILE, :]
    c2c[...] = cs2[TILE - 1:TILE, :]
    z1c[...] = zc1[TILE - 1:TILE, :]
    z2c[...] = zc2[TILE - 1:TILE, :]

    @pl.when(i == NT - 1)
    def _fin():
        ones = jnp.ones((1, TILE), jnp.float32)
        tot_ref[...] = jnp.concatenate(
            [cs1[TILE - 1:TILE, :], cs2[TILE - 1:TILE, :],
             zc1[TILE - 1:TILE, :] * ones, zc2[TILE - 1:TILE, :] * ones,
             jnp.zeros((4, TILE), jnp.float32)], axis=0)


_cumsum = pl.pallas_call(
    _cumsum_body,
    grid=(NT,),
    in_specs=[
        pl.BlockSpec((1, 1), lambda i: (0, 0)),
        pl.BlockSpec((TILE, 1), lambda i: (i, 0)),
        pl.BlockSpec((TILE, F), lambda i: (i, 0)),
    ],
    out_specs=[
        pl.BlockSpec((TILE, F), lambda i: (i, 0)),
        pl.BlockSpec((TILE, F), lambda i: (i, 0)),
        pl.BlockSpec((TILE, 1), lambda i: (i, 0)),
        pl.BlockSpec((TILE, 1), lambda i: (i, 0)),
        pl.BlockSpec((8, TILE), lambda i: (0, 0)),
    ],
    out_shape=[
        jax.ShapeDtypeStruct((NP, F), jnp.float32),
        jax.ShapeDtypeStruct((NP, F), jnp.float32),
        jax.ShapeDtypeStruct((NP, 1), jnp.float32),
        jax.ShapeDtypeStruct((NP, 1), jnp.float32),
        jax.ShapeDtypeStruct((8, TILE), jnp.float32),
    ],
    scratch_shapes=[
        pltpu.VMEM((1, F), jnp.float32),
        pltpu.VMEM((1, F), jnp.float32),
        pltpu.VMEM((1, 1), jnp.float32),
        pltpu.VMEM((1, 1), jnp.float32),
    ],
)


@functools.partial(
    pl.kernel,
    mesh=_mesh,
    out_type=[
        jax.ShapeDtypeStruct((NP, F), jnp.float32),
        jax.ShapeDtypeStruct((NP, F), jnp.float32),
        jax.ShapeDtypeStruct((NP,), jnp.float32),
        jax.ShapeDtypeStruct((NP,), jnp.float32),
    ],
    scratch_types=[
        pltpu.VMEM((NP,), jnp.float32),
        pltpu.VMEM((NP,), jnp.float32),
        pltpu.VMEM((NIDX,), jnp.int32),
        pltpu.VMEM((NIDX,), jnp.int32),
        pltpu.VMEM((NIDX,), jnp.int32),
        pltpu.VMEM((NIDX,), jnp.int32),
        pltpu.VMEM((CHUNK, F), jnp.float32),
        pltpu.VMEM((CHUNK, F), jnp.float32),
        pltpu.VMEM((CHUNK,), jnp.float32),
        pltpu.VMEM((CHUNK,), jnp.float32),
        pltpu.SemaphoreType.DMA,
    ],
)
def _sc_pick(g_hbm, z1_hbm, z2_hbm, c1_hbm, c2_hbm,
             s1_hbm, s2_hbm, z1g_hbm, z2g_hbm,
             z1_v, z2_v, g0, g1, g2, g3, rows1_v, rows2_v, zo1_v, zo2_v, sem):
    wid = lax.axis_index("s") * 2 + lax.axis_index("c")
    base = wid * CHUNK
    gbufs = (g0, g1, g2, g3)
    for c in range(NCH):
        pltpu.sync_copy(g_hbm.at[pl.ds(base + c * NIDX, NIDX)], gbufs[c])
    cps = []
    for c in range(NCH):
        cps.append(pltpu.async_copy(
            c1_hbm.at[gbufs[c]], rows1_v.at[pl.ds(c * NIDX, NIDX)], sem))
        cps.append(pltpu.async_copy(
            c2_hbm.at[gbufs[c]], rows2_v.at[pl.ds(c * NIDX, NIDX)], sem))
    pltpu.sync_copy(z1_hbm, z1_v)
    pltpu.sync_copy(z2_hbm, z2_v)
    for c in range(NCH):
        for l in range(NIDX // L):
            gv = gbufs[c][pl.ds(l * L, L)]
            zo1_v[pl.ds(c * NIDX + l * L, L)] = plsc.load_gather(z1_v, [gv])
            zo2_v[pl.ds(c * NIDX + l * L, L)] = plsc.load_gather(z2_v, [gv])
    pltpu.sync_copy(zo1_v, z1g_hbm.at[pl.ds(base, CHUNK)])
    pltpu.sync_copy(zo2_v, z2g_hbm.at[pl.ds(base, CHUNK)])
    for cp in cps:
        cp.wait()
    pltpu.sync_copy(rows1_v, s1_hbm.at[pl.ds(base, CHUNK)])
    pltpu.sync_copy(rows2_v, s2_hbm.at[pl.ds(base, CHUNK)])


def _combine_body(dmax_ref, tot_ref, gamma_ref, beta_ref, s_ref,
                  z1g_ref, z2g_ref, s1_ref, s2_ref, out_ref, sum_s, sq_s):
    p = pl.program_id(0)
    j = pl.program_id(1)

    @pl.when((p == 0) & (j == 0))
    def _init():
        sum_s[...] = jnp.zeros_like(sum_s)
        sq_s[...] = jnp.zeros_like(sq_s)

    sd = s_ref[...] + dmax_ref[...]
    m = jnp.where(sd > 0, sd, 0.2 * sd)
    f1 = jnp.exp(sd - m)
    f2 = jnp.exp(0.2 * sd - m)
    t1v = tot_ref[0:1, :]
    t1s = tot_ref[2:3, 0:1]
    b1 = t1v - s1_ref[...]
    b2 = s2_ref[...]
    y1 = t1s - z1g_ref[...]
    y2 = z2g_ref[...]
    den = f1 * y1 + f2 * y2
    o = (f1 * b1 + f2 * b2) / den
    rid = lax.broadcasted_iota(jnp.int32, (T2, 1), 0) + j * T2
    valid = rid < N

    @pl.when(p == 0)
    def _acc():
        om = jnp.where(valid, o, 0.0)
        sum_s[...] += jnp.sum(om, axis=0, keepdims=True)
        sq_s[...] += jnp.sum(om * om, axis=0, keepdims=True)

    @pl.when(p == 1)
    def _wr():
        mean = sum_s[...] * (1.0 / N)
        var = sq_s[...] * (1.0 / N) - mean * mean
        inv = lax.rsqrt(var + 1e-5)
        out_ref[...] = jnp.maximum(
            (o - mean) * inv * gamma_ref[...] + beta_ref[...], 0.0)


_combine = pl.pallas_call(
    _combine_body,
    grid=(2, NT2),
    in_specs=[
        pl.BlockSpec((1, 1), lambda p, j: (0, 0)),
        pl.BlockSpec((8, TILE), lambda p, j: (0, 0)),
        pl.BlockSpec((1, F), lambda p, j: (0, 0)),
        pl.BlockSpec((1, F), lambda p, j: (0, 0)),
        pl.BlockSpec((T2, 1), lambda p, j: (j, 0)),
        pl.BlockSpec((T2, 1), lambda p, j: (j, 0)),
        pl.BlockSpec((T2, 1), lambda p, j: (j, 0)),
        pl.BlockSpec((T2, F), lambda p, j: (j, 0)),
        pl.BlockSpec((T2, F), lambda p, j: (j, 0)),
    ],
    out_specs=pl.BlockSpec((T2, F), lambda p, j: (j, 0)),
    out_shape=jax.ShapeDtypeStruct((NP, F), jnp.float32),
    scratch_shapes=[
        pltpu.VMEM((1, F), jnp.float32),
        pltpu.VMEM((1, F), jnp.float32),
    ],
)


def kernel(x, graph, W, a_src, a_dst, gamma, beta):
    del graph
    h, s, d = _proj(x, W, a_src.reshape(1, F), a_dst.reshape(1, F))
    sortd, perm = lax.sort_key_val(d.reshape(N), jnp.arange(N, dtype=jnp.int32))
    ds_pad = jnp.concatenate([jnp.full((PAD,), NEG, jnp.float32), sortd])
    p_pad = jnp.concatenate([jnp.zeros((PAD,), jnp.int32), perm])
    s_pad = jnp.concatenate([s.reshape(N), jnp.zeros((PAD,), jnp.float32)])
    dmax = sortd[N - 1:N].reshape(1, 1)
    hp, g = _sc_rank(p_pad, s_pad, ds_pad, h)
    c1, c2, z1, z2, tot = _cumsum(dmax, ds_pad.reshape(NP, 1), hp)
    s1g, s2g, z1g, z2g = _sc_pick(g, z1.reshape(NP), z2.reshape(NP), c1, c2)
    o = _combine(dmax, tot, gamma.reshape(1, F), beta.reshape(1, F),
                 s_pad.reshape(NP, 1), z1g.reshape(NP, 1), z2g.reshape(NP, 1),
                 s1g, s2g)
    return o[:N]


# speedup vs baseline: 1.1045x; 1.1045x over previous
import functools

import jax
import jax.numpy as jnp
from jax import lax
from jax.experimental import pallas as pl
from jax.experimental.pallas import tpu as pltpu
from jax.experimental.pallas import tpu_sc as plsc

N = 10000
F = 128
NW = 32
CHUNK = 320
NP = NW * CHUNK
PAD = NP - N
NEG = -1e30
NIDX = 80
NCH = CHUNK // NIDX
ZL = 128
TILE = 128
NT = NP // TILE
T2 = 512
NT2 = NP // T2


def _proj_body(x_ref, w_ref, asrc_ref, adst_ref, h_ref, s_ref, d_ref):
    h = jnp.dot(x_ref[...], w_ref[...], preferred_element_type=jnp.float32)
    h_ref[...] = h
    s_ref[...] = jnp.sum(h * asrc_ref[...], axis=1, keepdims=True)
    d_ref[...] = jnp.sum(h * adst_ref[...], axis=1, keepdims=True)


_proj = pl.pallas_call(
    _proj_body,
    out_shape=[
        jax.ShapeDtypeStruct((N, F), jnp.float32),
        jax.ShapeDtypeStruct((N, 1), jnp.float32),
        jax.ShapeDtypeStruct((N, 1), jnp.float32),
    ],
)


@functools.cache
def _make_sc_hgather():
    mesh = plsc.VectorSubcoreMesh(core_axis_name="c", subcore_axis_name="s")
    return functools.partial(
        pl.kernel,
        mesh=mesh,
        out_type=jax.ShapeDtypeStruct((NP, F), jnp.float32),
        scratch_types=[
            pltpu.VMEM((NIDX,), jnp.int32),
            pltpu.VMEM((NIDX,), jnp.int32),
            pltpu.VMEM((NIDX,), jnp.int32),
            pltpu.VMEM((NIDX,), jnp.int32),
            pltpu.VMEM((CHUNK, F), jnp.float32),
            pltpu.SemaphoreType.DMA,
        ],
    )(_sc_hgather_body)


def _sc_hgather_body(p_hbm, h_hbm, hp_hbm, p0, p1, p2, p3, rows_v, sem):
    wid = lax.axis_index("s") * 2 + lax.axis_index("c")
    base = wid * CHUNK
    pbufs = (p0, p1, p2, p3)
    for c in range(NCH):
        pltpu.sync_copy(p_hbm.at[pl.ds(base + c * NIDX, NIDX)], pbufs[c])
    cps = []
    for c in range(NCH):
        cps.append(pltpu.async_copy(
            h_hbm.at[pbufs[c]], rows_v.at[pl.ds(c * NIDX, NIDX)], sem))
    for cp in cps:
        cp.wait()
    pltpu.sync_copy(rows_v, hp_hbm.at[pl.ds(base, CHUNK)])


def _cumsum_body(dmax_ref, ds_ref, hp_ref, c1_ref, c2_ref, zz_ref,
                 tot_ref, c1c, c2c, z1c, z2c):
    i = pl.program_id(0)

    @pl.when(i == 0)
    def _init():
        c1c[...] = jnp.zeros_like(c1c)
        c2c[...] = jnp.zeros_like(c2c)
        z1c[...] = jnp.zeros_like(z1c)
        z2c[...] = jnp.zeros_like(z2c)

    rid = lax.broadcasted_iota(jnp.int32, (TILE, TILE), 0)
    cid = lax.broadcasted_iota(jnp.int32, (TILE, TILE), 1)
    lt = jnp.where(rid >= cid, 1.0, 0.0).astype(jnp.float32)
    dcol = ds_ref[...]
    dmax = dmax_ref[...]
    w1 = jnp.exp(dcol - dmax)
    w2 = jnp.exp(0.2 * (dcol - dmax))
    tile = hp_ref[...]
    cs1 = jnp.dot(lt, w1 * tile, preferred_element_type=jnp.float32) + c1c[...]
    cs2 = jnp.dot(lt, w2 * tile, preferred_element_type=jnp.float32) + c2c[...]
    zc1 = jnp.dot(lt, w1, preferred_element_type=jnp.float32) + z1c[...]
    zc2 = jnp.dot(lt, w2, preferred_element_type=jnp.float32) + z2c[...]
    c1_ref[...] = cs1
    c2_ref[...] = cs2
    zz_ref[...] = jnp.concatenate(
        [zc1, zc2, jnp.zeros((TILE, ZL - 2), jnp.float32)], axis=1)
    c1c[...] = cs1[TILE - 1:TILE, :]
    c2c[...] = cs2[TILE - 1:TILE, :]
    z1c[...] = zc1[TILE - 1:TILE, :]
    z2c[...] = zc2[TILE - 1:TILE, :]

    @pl.when(i == NT - 1)
    def _fin():
        ones = jnp.ones((1, TILE), jnp.float32)
        tot_ref[...] = jnp.concatenate(
            [cs1[TILE - 1:TILE, :], cs2[TILE - 1:TILE, :],
             zc1[TILE - 1:TILE, :] * ones, zc2[TILE - 1:TILE, :] * ones,
             jnp.zeros((4, TILE), jnp.float32)], axis=0)


_cumsum = pl.pallas_call(
    _cumsum_body,
    grid=(NT,),
    in_specs=[
        pl.BlockSpec((1, 1), lambda i: (0, 0)),
        pl.BlockSpec((TILE, 1), lambda i: (i, 0)),
        pl.BlockSpec((TILE, F), lambda i: (i, 0)),
    ],
    out_specs=[
        pl.BlockSpec((TILE, F), lambda i: (i, 0)),
        pl.BlockSpec((TILE, F), lambda i: (i, 0)),
        pl.BlockSpec((TILE, ZL), lambda i: (i, 0)),
        pl.BlockSpec((8, TILE), lambda i: (0, 0)),
    ],
    out_shape=[
        jax.ShapeDtypeStruct((NP, F), jnp.float32),
        jax.ShapeDtypeStruct((NP, F), jnp.float32),
        jax.ShapeDtypeStruct((NP, ZL), jnp.float32),
        jax.ShapeDtypeStruct((8, TILE), jnp.float32),
    ],
    scratch_shapes=[
        pltpu.VMEM((1, F), jnp.float32),
        pltpu.VMEM((1, F), jnp.float32),
        pltpu.VMEM((1, 1), jnp.float32),
        pltpu.VMEM((1, 1), jnp.float32),
    ],
)


@functools.cache
def _make_sc_pick():
    mesh = plsc.VectorSubcoreMesh(core_axis_name="c", subcore_axis_name="s")
    return functools.partial(
        pl.kernel,
        mesh=mesh,
        out_type=[
            jax.ShapeDtypeStruct((NP, F), jnp.float32),
            jax.ShapeDtypeStruct((NP, F), jnp.float32),
            jax.ShapeDtypeStruct((NP, ZL), jnp.float32),
        ],
        scratch_types=[
            pltpu.VMEM((NIDX,), jnp.int32),
            pltpu.VMEM((NIDX,), jnp.int32),
            pltpu.VMEM((NIDX,), jnp.int32),
            pltpu.VMEM((NIDX,), jnp.int32),
            pltpu.VMEM((CHUNK, F), jnp.float32),
            pltpu.VMEM((CHUNK, F), jnp.float32),
            pltpu.VMEM((CHUNK, ZL), jnp.float32),
            pltpu.SemaphoreType.DMA,
        ],
    )(_sc_pick_body)


def _sc_pick_body(g_hbm, c1_hbm, c2_hbm, zz_hbm,
                  s1_hbm, s2_hbm, zg_hbm,
                  g0, g1, g2, g3, rows1_v, rows2_v, rowsz_v, sem):
    wid = lax.axis_index("s") * 2 + lax.axis_index("c")
    base = wid * CHUNK
    gbufs = (g0, g1, g2, g3)
    for c in range(NCH):
        pltpu.sync_copy(g_hbm.at[pl.ds(base + c * NIDX, NIDX)], gbufs[c])
    cps = []
    for c in range(NCH):
        cps.append(pltpu.async_copy(
            c1_hbm.at[gbufs[c]], rows1_v.at[pl.ds(c * NIDX, NIDX)], sem))
        cps.append(pltpu.async_copy(
            c2_hbm.at[gbufs[c]], rows2_v.at[pl.ds(c * NIDX, NIDX)], sem))
        cps.append(pltpu.async_copy(
            zz_hbm.at[gbufs[c]], rowsz_v.at[pl.ds(c * NIDX, NIDX)], sem))
    for cp in cps:
        cp.wait()
    pltpu.sync_copy(rows1_v, s1_hbm.at[pl.ds(base, CHUNK)])
    pltpu.sync_copy(rows2_v, s2_hbm.at[pl.ds(base, CHUNK)])
    pltpu.sync_copy(rowsz_v, zg_hbm.at[pl.ds(base, CHUNK)])


def _combine_body(dmax_ref, tot_ref, gamma_ref, beta_ref, s_ref,
                  zg_ref, s1_ref, s2_ref, out_ref, sum_s, sq_s):
    p = pl.program_id(0)
    j = pl.program_id(1)

    @pl.when((p == 0) & (j == 0))
    def _init():
        sum_s[...] = jnp.zeros_like(sum_s)
        sq_s[...] = jnp.zeros_like(sq_s)

    sd = s_ref[...] + dmax_ref[...]
    m = jnp.where(sd > 0, sd, 0.2 * sd)
    f1 = jnp.exp(sd - m)
    f2 = jnp.exp(0.2 * sd - m)
    t1v = tot_ref[0:1, :]
    t1s = tot_ref[2:3, 0:1]
    b1 = t1v - s1_ref[...]
    b2 = s2_ref[...]
    y1 = t1s - zg_ref[:, 0:1]
    y2 = zg_ref[:, 1:2]
    den = f1 * y1 + f2 * y2
    o = (f1 * b1 + f2 * b2) / den
    rid = lax.broadcasted_iota(jnp.int32, (T2, 1), 0) + j * T2
    valid = rid < N

    @pl.when(p == 0)
    def _acc():
        om = jnp.where(valid, o, 0.0)
        sum_s[...] += jnp.sum(om, axis=0, keepdims=True)
        sq_s[...] += jnp.sum(om * om, axis=0, keepdims=True)

    @pl.when(p == 1)
    def _wr():
        mean = sum_s[...] * (1.0 / N)
        var = sq_s[...] * (1.0 / N) - mean * mean
        inv = lax.rsqrt(var + 1e-5)
        out_ref[...] = jnp.maximum(
            (o - mean) * inv * gamma_ref[...] + beta_ref[...], 0.0)


_combine = pl.pallas_call(
    _combine_body,
    grid=(2, NT2),
    in_specs=[
        pl.BlockSpec((1, 1), lambda p, j: (0, 0)),
        pl.BlockSpec((8, TILE), lambda p, j: (0, 0)),
        pl.BlockSpec((1, F), lambda p, j: (0, 0)),
        pl.BlockSpec((1, F), lambda p, j: (0, 0)),
        pl.BlockSpec((T2, 1), lambda p, j: (j, 0)),
        pl.BlockSpec((T2, ZL), lambda p, j: (j, 0)),
        pl.BlockSpec((T2, F), lambda p, j: (j, 0)),
        pl.BlockSpec((T2, F), lambda p, j: (j, 0)),
    ],
    out_specs=pl.BlockSpec((T2, F), lambda p, j: (j, 0)),
    out_shape=jax.ShapeDtypeStruct((NP, F), jnp.float32),
    scratch_shapes=[
        pltpu.VMEM((1, F), jnp.float32),
        pltpu.VMEM((1, F), jnp.float32),
    ],
)


def kernel(x, graph, W, a_src, a_dst, gamma, beta):
    del graph
    h, s, d = _proj(x, W, a_src.reshape(1, F), a_dst.reshape(1, F))
    sortd, perm = lax.sort_key_val(d.reshape(N), jnp.arange(N, dtype=jnp.int32))
    ds_pad = jnp.concatenate([jnp.full((PAD,), NEG, jnp.float32), sortd])
    p_pad = jnp.concatenate([jnp.zeros((PAD,), jnp.int32), perm])
    s_pad = jnp.concatenate([s.reshape(N), jnp.zeros((PAD,), jnp.float32)])
    dmax = sortd[N - 1:N].reshape(1, 1)
    k = jnp.searchsorted(ds_pad, -s_pad, side="right")
    g = jnp.maximum(k - 1, 0).astype(jnp.int32)
    hp = _make_sc_hgather()(p_pad, h)
    c1, c2, zz, tot = _cumsum(dmax, ds_pad.reshape(NP, 1), hp)
    s1g, s2g, zg = _make_sc_pick()(g, c1, c2, zz)
    o = _combine(dmax, tot, gamma.reshape(1, F), beta.reshape(1, F),
                 s_pad.reshape(NP, 1), zg, s1g, s2g)
    return o[:N]


# speedup vs baseline: 2.1006x; 1.9018x over previous
import functools

import jax
import jax.numpy as jnp
from jax import lax
from jax.experimental import pallas as pl
from jax.experimental.pallas import tpu as pltpu
from jax.experimental.pallas import tpu_sc as plsc

N = 10000
F = 128
NW = 32
CHUNK = 320
NP = NW * CHUNK
PAD = NP - N
NEG = -1e30
NIDX = 80
NCH = CHUNK // NIDX
ZL = 128
TILE = 128
NT = NP // TILE
NB = NP // 128
T2 = 512
NT2 = NP // T2


def _proj_body(x_ref, w_ref, asrc_ref, adst_ref, h_ref, s_ref, d_ref):
    h = jnp.dot(x_ref[...], w_ref[...], preferred_element_type=jnp.float32)
    h_ref[...] = h
    s_ref[...] = jnp.sum(h * asrc_ref[...], axis=1, keepdims=True)
    d_ref[...] = jnp.sum(h * adst_ref[...], axis=1, keepdims=True)


_proj = pl.pallas_call(
    _proj_body,
    out_shape=[
        jax.ShapeDtypeStruct((N, F), jnp.float32),
        jax.ShapeDtypeStruct((N, 1), jnp.float32),
        jax.ShapeDtypeStruct((N, 1), jnp.float32),
    ],
)


@functools.cache
def _make_sc_hgather():
    mesh = plsc.VectorSubcoreMesh(core_axis_name="c", subcore_axis_name="s")
    return functools.partial(
        pl.kernel,
        mesh=mesh,
        out_type=jax.ShapeDtypeStruct((NP, F), jnp.float32),
        scratch_types=[
            pltpu.VMEM((NIDX,), jnp.int32),
            pltpu.VMEM((NIDX,), jnp.int32),
            pltpu.VMEM((NIDX,), jnp.int32),
            pltpu.VMEM((NIDX,), jnp.int32),
            pltpu.VMEM((CHUNK, F), jnp.float32),
            pltpu.SemaphoreType.DMA,
        ],
    )(_sc_hgather_body)


def _sc_hgather_body(p_hbm, h_hbm, hp_hbm, p0, p1, p2, p3, rows_v, sem):
    wid = lax.axis_index("s") * 2 + lax.axis_index("c")
    base = wid * CHUNK
    pbufs = (p0, p1, p2, p3)
    for c in range(NCH):
        pltpu.sync_copy(p_hbm.at[pl.ds(base + c * NIDX, NIDX)], pbufs[c])
    cps = []
    for c in range(NCH):
        cps.append(pltpu.async_copy(
            h_hbm.at[pbufs[c]], rows_v.at[pl.ds(c * NIDX, NIDX)], sem))
    for cp in cps:
        cp.wait()
    pltpu.sync_copy(rows_v, hp_hbm.at[pl.ds(base, CHUNK)])


def _cumsum_body(dmax_ref, ds_ref, hp_ref, s_ref, ds3_ref, pv_ref,
                 c1_ref, c2_ref, zz_ref, tot_ref, g_ref,
                 c1c, c2c, z1c, z2c):
    i = pl.program_id(0)

    @pl.when(i == 0)
    def _init():
        c1c[...] = jnp.zeros_like(c1c)
        c2c[...] = jnp.zeros_like(c2c)
        z1c[...] = jnp.zeros_like(z1c)
        z2c[...] = jnp.zeros_like(z2c)

    rid = lax.broadcasted_iota(jnp.int32, (TILE, TILE), 0)
    cid = lax.broadcasted_iota(jnp.int32, (TILE, TILE), 1)
    lt = jnp.where(rid >= cid, 1.0, 0.0).astype(jnp.float32)
    dcol = ds_ref[...]
    dmax = dmax_ref[...]
    w1 = jnp.exp(dcol - dmax)
    w2 = jnp.exp(0.2 * (dcol - dmax))
    tile = hp_ref[...]
    cs1 = jnp.dot(lt, w1 * tile, preferred_element_type=jnp.float32) + c1c[...]
    cs2 = jnp.dot(lt, w2 * tile, preferred_element_type=jnp.float32) + c2c[...]
    zc1 = jnp.dot(lt, w1, preferred_element_type=jnp.float32) + z1c[...]
    zc2 = jnp.dot(lt, w2, preferred_element_type=jnp.float32) + z2c[...]
    c1_ref[...] = cs1
    c2_ref[...] = cs2
    zz_ref[...] = jnp.concatenate(
        [zc1, zc2, jnp.zeros((TILE, ZL - 2), jnp.float32)], axis=1)
    c1c[...] = cs1[TILE - 1:TILE, :]
    c2c[...] = cs2[TILE - 1:TILE, :]
    z1c[...] = zc1[TILE - 1:TILE, :]
    z2c[...] = zc2[TILE - 1:TILE, :]

    t = -s_ref[...]
    b = jnp.sum((pv_ref[...] <= t).astype(jnp.float32),
                axis=1, keepdims=True) - 1.0
    cid80 = lax.broadcasted_iota(jnp.int32, (TILE, NB), 1)
    onehot = (cid80 == b.astype(jnp.int32)).astype(jnp.float32)
    brow = jnp.dot(onehot, ds3_ref[...],
                   preferred_element_type=jnp.float32)
    c = jnp.sum((brow <= t).astype(jnp.float32), axis=1, keepdims=True)
    g_ref[...] = jnp.maximum(
        b.astype(jnp.int32) * 128 + c.astype(jnp.int32) - 1, 0)

    @pl.when(i == NT - 1)
    def _fin():
        ones = jnp.ones((1, TILE), jnp.float32)
        tot_ref[...] = jnp.concatenate(
            [cs1[TILE - 1:TILE, :], cs2[TILE - 1:TILE, :],
             zc1[TILE - 1:TILE, :] * ones, zc2[TILE - 1:TILE, :] * ones,
             jnp.zeros((4, TILE), jnp.float32)], axis=0)


_cumsum = pl.pallas_call(
    _cumsum_body,
    grid=(NT,),
    in_specs=[
        pl.BlockSpec((1, 1), lambda i: (0, 0)),
        pl.BlockSpec((TILE, 1), lambda i: (i, 0)),
        pl.BlockSpec((TILE, F), lambda i: (i, 0)),
        pl.BlockSpec((TILE, 1), lambda i: (i, 0)),
        pl.BlockSpec((NB, 128), lambda i: (0, 0)),
        pl.BlockSpec((1, 128), lambda i: (0, 0)),
    ],
    out_specs=[
        pl.BlockSpec((TILE, F), lambda i: (i, 0)),
        pl.BlockSpec((TILE, F), lambda i: (i, 0)),
        pl.BlockSpec((TILE, ZL), lambda i: (i, 0)),
        pl.BlockSpec((8, TILE), lambda i: (0, 0)),
        pl.BlockSpec((TILE, 1), lambda i: (i, 0)),
    ],
    out_shape=[
        jax.ShapeDtypeStruct((NP, F), jnp.float32),
        jax.ShapeDtypeStruct((NP, F), jnp.float32),
        jax.ShapeDtypeStruct((NP, ZL), jnp.float32),
        jax.ShapeDtypeStruct((8, TILE), jnp.float32),
        jax.ShapeDtypeStruct((NP, 1), jnp.int32),
    ],
    scratch_shapes=[
        pltpu.VMEM((1, F), jnp.float32),
        pltpu.VMEM((1, F), jnp.float32),
        pltpu.VMEM((1, 1), jnp.float32),
        pltpu.VMEM((1, 1), jnp.float32),
    ],
)


@functools.cache
def _make_sc_pick():
    mesh = plsc.VectorSubcoreMesh(core_axis_name="c", subcore_axis_name="s")
    return functools.partial(
        pl.kernel,
        mesh=mesh,
        out_type=[
            jax.ShapeDtypeStruct((NP, F), jnp.float32),
            jax.ShapeDtypeStruct((NP, F), jnp.float32),
            jax.ShapeDtypeStruct((NP, ZL), jnp.float32),
        ],
        scratch_types=[
            pltpu.VMEM((NIDX,), jnp.int32),
            pltpu.VMEM((NIDX,), jnp.int32),
            pltpu.VMEM((NIDX,), jnp.int32),
            pltpu.VMEM((NIDX,), jnp.int32),
            pltpu.VMEM((CHUNK, F), jnp.float32),
            pltpu.VMEM((CHUNK, F), jnp.float32),
            pltpu.VMEM((CHUNK, ZL), jnp.float32),
            pltpu.SemaphoreType.DMA,
        ],
    )(_sc_pick_body)


def _sc_pick_body(g_hbm, c1_hbm, c2_hbm, zz_hbm,
                  s1_hbm, s2_hbm, zg_hbm,
                  g0, g1, g2, g3, rows1_v, rows2_v, rowsz_v, sem):
    wid = lax.axis_index("s") * 2 + lax.axis_index("c")
    base = wid * CHUNK
    gbufs = (g0, g1, g2, g3)
    for c in range(NCH):
        pltpu.sync_copy(g_hbm.at[pl.ds(base + c * NIDX, NIDX)], gbufs[c])
    cps = []
    for c in range(NCH):
        cps.append(pltpu.async_copy(
            c1_hbm.at[gbufs[c]], rows1_v.at[pl.ds(c * NIDX, NIDX)], sem))
        cps.append(pltpu.async_copy(
            c2_hbm.at[gbufs[c]], rows2_v.at[pl.ds(c * NIDX, NIDX)], sem))
        cps.append(pltpu.async_copy(
            zz_hbm.at[gbufs[c]], rowsz_v.at[pl.ds(c * NIDX, NIDX)], sem))
    for cp in cps:
        cp.wait()
    pltpu.sync_copy(rows1_v, s1_hbm.at[pl.ds(base, CHUNK)])
    pltpu.sync_copy(rows2_v, s2_hbm.at[pl.ds(base, CHUNK)])
    pltpu.sync_copy(rowsz_v, zg_hbm.at[pl.ds(base, CHUNK)])


def _combine_body(dmax_ref, tot_ref, gamma_ref, beta_ref, s_ref,
                  zg_ref, s1_ref, s2_ref, out_ref, sum_s, sq_s):
    p = pl.program_id(0)
    j = pl.program_id(1)

    @pl.when((p == 0) & (j == 0))
    def _init():
        sum_s[...] = jnp.zeros_like(sum_s)
        sq_s[...] = jnp.zeros_like(sq_s)

    sd = s_ref[...] + dmax_ref[...]
    m = jnp.where(sd > 0, sd, 0.2 * sd)
    f1 = jnp.exp(sd - m)
    f2 = jnp.exp(0.2 * sd - m)
    t1v = tot_ref[0:1, :]
    t1s = tot_ref[2:3, 0:1]
    b1 = t1v - s1_ref[...]
    b2 = s2_ref[...]
    y1 = t1s - zg_ref[:, 0:1]
    y2 = zg_ref[:, 1:2]
    den = f1 * y1 + f2 * y2
    o = (f1 * b1 + f2 * b2) / den
    rid = lax.broadcasted_iota(jnp.int32, (T2, 1), 0) + j * T2
    valid = rid < N

    @pl.when(p == 0)
    def _acc():
        om = jnp.where(valid, o, 0.0)
        sum_s[...] += jnp.sum(om, axis=0, keepdims=True)
        sq_s[...] += jnp.sum(om * om, axis=0, keepdims=True)

    @pl.when(p == 1)
    def _wr():
        mean = sum_s[...] * (1.0 / N)
        var = sq_s[...] * (1.0 / N) - mean * mean
        inv = lax.rsqrt(var + 1e-5)
        out_ref[...] = jnp.maximum(
            (o - mean) * inv * gamma_ref[...] + beta_ref[...], 0.0)


_combine = pl.pallas_call(
    _combine_body,
    grid=(2, NT2),
    in_specs=[
        pl.BlockSpec((1, 1), lambda p, j: (0, 0)),
        pl.BlockSpec((8, TILE), lambda p, j: (0, 0)),
        pl.BlockSpec((1, F), lambda p, j: (0, 0)),
        pl.BlockSpec((1, F), lambda p, j: (0, 0)),
        pl.BlockSpec((T2, 1), lambda p, j: (j, 0)),
        pl.BlockSpec((T2, ZL), lambda p, j: (j, 0)),
        pl.BlockSpec((T2, F), lambda p, j: (j, 0)),
        pl.BlockSpec((T2, F), lambda p, j: (j, 0)),
    ],
    out_specs=pl.BlockSpec((T2, F), lambda p, j: (j, 0)),
    out_shape=jax.ShapeDtypeStruct((NP, F), jnp.float32),
    scratch_shapes=[
        pltpu.VMEM((1, F), jnp.float32),
        pltpu.VMEM((1, F), jnp.float32),
    ],
)


def kernel(x, graph, W, a_src, a_dst, gamma, beta):
    del graph
    h, s, d = _proj(x, W, a_src.reshape(1, F), a_dst.reshape(1, F))
    sortd, perm = lax.sort_key_val(d.reshape(N), jnp.arange(N, dtype=jnp.int32))
    ds_pad = jnp.concatenate([jnp.full((PAD,), NEG, jnp.float32), sortd])
    p_pad = jnp.concatenate([jnp.zeros((PAD,), jnp.int32), perm])
    s_pad = jnp.concatenate([s.reshape(N), jnp.zeros((PAD,), jnp.float32)])
    dmax = sortd[N - 1:N].reshape(1, 1)
    pv = jnp.concatenate(
        [ds_pad[::128], jnp.full((128 - NB,), jnp.inf, jnp.float32)]
    ).reshape(1, 128)
    hp = _make_sc_hgather()(p_pad, h)
    c1, c2, zz, tot, g = _cumsum(
        dmax, ds_pad.reshape(NP, 1), hp, s_pad.reshape(NP, 1),
        ds_pad.reshape(NB, 128), pv)
    s1g, s2g, zg = _make_sc_pick()(g.reshape(NP), c1, c2, zz)
    o = _combine(dmax, tot, gamma.reshape(1, F), beta.reshape(1, F),
                 s_pad.reshape(NP, 1), zg, s1g, s2g)
    return o[:N]


# speedup vs baseline: 2.4385x; 1.1608x over previous
import functools

import jax
import jax.numpy as jnp
from jax import lax
from jax.experimental import pallas as pl
from jax.experimental.pallas import tpu as pltpu
from jax.experimental.pallas import tpu_sc as plsc

N = 10000
F = 128
NW = 32
CHUNK = 320
NP = NW * CHUNK
PAD = NP - N
NEG = -1e30
NIDX = 80
NCH = CHUNK // NIDX
ZL = 128
TILE = 512
NT = NP // TILE
NB = NP // 128
T2 = 512
NT2 = NP // T2


def _proj_body(x_ref, w_ref, asrc_ref, adst_ref, h_ref, s_ref, d_ref):
    h = jnp.dot(x_ref[...], w_ref[...], preferred_element_type=jnp.float32)
    h_ref[...] = h
    s_ref[...] = jnp.sum(h * asrc_ref[...], axis=1, keepdims=True)
    d_ref[...] = jnp.sum(h * adst_ref[...], axis=1, keepdims=True)


_proj = pl.pallas_call(
    _proj_body,
    out_shape=[
        jax.ShapeDtypeStruct((N, F), jnp.float32),
        jax.ShapeDtypeStruct((N, 1), jnp.float32),
        jax.ShapeDtypeStruct((N, 1), jnp.float32),
    ],
)


@functools.cache
def _make_sc_hgather():
    mesh = plsc.VectorSubcoreMesh(core_axis_name="c", subcore_axis_name="s")
    return functools.partial(
        pl.kernel,
        mesh=mesh,
        out_type=jax.ShapeDtypeStruct((NP, F), jnp.float32),
        scratch_types=[
            pltpu.VMEM((NIDX,), jnp.int32),
            pltpu.VMEM((NIDX,), jnp.int32),
            pltpu.VMEM((NIDX,), jnp.int32),
            pltpu.VMEM((NIDX,), jnp.int32),
            pltpu.VMEM((CHUNK, F), jnp.float32),
            pltpu.SemaphoreType.DMA,
        ],
    )(_sc_hgather_body)


def _sc_hgather_body(p_hbm, h_hbm, hp_hbm, p0, p1, p2, p3, rows_v, sem):
    wid = lax.axis_index("s") * 2 + lax.axis_index("c")
    base = wid * CHUNK
    pbufs = (p0, p1, p2, p3)
    for c in range(NCH):
        pltpu.sync_copy(p_hbm.at[pl.ds(base + c * NIDX, NIDX)], pbufs[c])
    cps = []
    for c in range(NCH):
        cps.append(pltpu.async_copy(
            h_hbm.at[pbufs[c]], rows_v.at[pl.ds(c * NIDX, NIDX)], sem))
    for cp in cps:
        cp.wait()
    pltpu.sync_copy(rows_v, hp_hbm.at[pl.ds(base, CHUNK)])


def _cumsum_body(dmax_ref, ds_ref, hp_ref, s_ref, ds3_ref, pv_ref,
                 c1_ref, c2_ref, zz_ref, tot_ref, g_ref,
                 c1c, c2c, z1c, z2c):
    i = pl.program_id(0)

    @pl.when(i == 0)
    def _init():
        c1c[...] = jnp.zeros_like(c1c)
        c2c[...] = jnp.zeros_like(c2c)
        z1c[...] = jnp.zeros_like(z1c)
        z2c[...] = jnp.zeros_like(z2c)

    rid = lax.broadcasted_iota(jnp.int32, (TILE, TILE), 0)
    cid = lax.broadcasted_iota(jnp.int32, (TILE, TILE), 1)
    lt = jnp.where(rid >= cid, 1.0, 0.0).astype(jnp.float32)
    dcol = ds_ref[...]
    dmax = dmax_ref[...]
    w1 = jnp.exp(dcol - dmax)
    w2 = jnp.exp(0.2 * (dcol - dmax))
    tile = hp_ref[...]
    cs1 = jnp.dot(lt, w1 * tile, preferred_element_type=jnp.float32) + c1c[...]
    cs2 = jnp.dot(lt, w2 * tile, preferred_element_type=jnp.float32) + c2c[...]
    zc1 = jnp.dot(lt, w1, preferred_element_type=jnp.float32) + z1c[...]
    zc2 = jnp.dot(lt, w2, preferred_element_type=jnp.float32) + z2c[...]
    c1_ref[...] = cs1
    c2_ref[...] = cs2
    zz_ref[...] = jnp.concatenate(
        [zc1, zc2, jnp.zeros((TILE, ZL - 2), jnp.float32)], axis=1)
    c1c[...] = cs1[TILE - 1:TILE, :]
    c2c[...] = cs2[TILE - 1:TILE, :]
    z1c[...] = zc1[TILE - 1:TILE, :]
    z2c[...] = zc2[TILE - 1:TILE, :]

    t = -s_ref[...]
    b = jnp.sum((pv_ref[...] <= t).astype(jnp.float32),
                axis=1, keepdims=True) - 1.0
    cid80 = lax.broadcasted_iota(jnp.int32, (TILE, NB), 1)
    onehot = (cid80 == b.astype(jnp.int32)).astype(jnp.float32)
    brow = jnp.dot(onehot, ds3_ref[...],
                   preferred_element_type=jnp.float32)
    c = jnp.sum((brow <= t).astype(jnp.float32), axis=1, keepdims=True)
    g_ref[...] = jnp.maximum(
        b.astype(jnp.int32) * 128 + c.astype(jnp.int32) - 1, 0)

    @pl.when(i == NT - 1)
    def _fin():
        ones = jnp.ones((1, F), jnp.float32)
        tot_ref[...] = jnp.concatenate(
            [cs1[TILE - 1:TILE, :], cs2[TILE - 1:TILE, :],
             zc1[TILE - 1:TILE, :] * ones, zc2[TILE - 1:TILE, :] * ones,
             jnp.zeros((4, F), jnp.float32)], axis=0)


_cumsum = pl.pallas_call(
    _cumsum_body,
    grid=(NT,),
    in_specs=[
        pl.BlockSpec((1, 1), lambda i: (0, 0)),
        pl.BlockSpec((TILE, 1), lambda i: (i, 0)),
        pl.BlockSpec((TILE, F), lambda i: (i, 0)),
        pl.BlockSpec((TILE, 1), lambda i: (i, 0)),
        pl.BlockSpec((NB, 128), lambda i: (0, 0)),
        pl.BlockSpec((1, 128), lambda i: (0, 0)),
    ],
    out_specs=[
        pl.BlockSpec((TILE, F), lambda i: (i, 0)),
        pl.BlockSpec((TILE, F), lambda i: (i, 0)),
        pl.BlockSpec((TILE, ZL), lambda i: (i, 0)),
        pl.BlockSpec((8, F), lambda i: (0, 0)),
        pl.BlockSpec((TILE, 1), lambda i: (i, 0)),
    ],
    out_shape=[
        jax.ShapeDtypeStruct((NP, F), jnp.float32),
        jax.ShapeDtypeStruct((NP, F), jnp.float32),
        jax.ShapeDtypeStruct((NP, ZL), jnp.float32),
        jax.ShapeDtypeStruct((8, F), jnp.float32),
        jax.ShapeDtypeStruct((NP, 1), jnp.int32),
    ],
    scratch_shapes=[
        pltpu.VMEM((1, F), jnp.float32),
        pltpu.VMEM((1, F), jnp.float32),
        pltpu.VMEM((1, 1), jnp.float32),
        pltpu.VMEM((1, 1), jnp.float32),
    ],
)


@functools.cache
def _make_sc_pick():
    mesh = plsc.VectorSubcoreMesh(core_axis_name="c", subcore_axis_name="s")
    return functools.partial(
        pl.kernel,
        mesh=mesh,
        out_type=[
            jax.ShapeDtypeStruct((NP, F), jnp.float32),
            jax.ShapeDtypeStruct((NP, F), jnp.float32),
            jax.ShapeDtypeStruct((NP, ZL), jnp.float32),
        ],
        scratch_types=[
            pltpu.VMEM((NIDX,), jnp.int32),
            pltpu.VMEM((NIDX,), jnp.int32),
            pltpu.VMEM((NIDX,), jnp.int32),
            pltpu.VMEM((NIDX,), jnp.int32),
            pltpu.VMEM((CHUNK, F), jnp.float32),
            pltpu.VMEM((CHUNK, F), jnp.float32),
            pltpu.VMEM((CHUNK, ZL), jnp.float32),
            pltpu.SemaphoreType.DMA,
        ],
    )(_sc_pick_body)


def _sc_pick_body(g_hbm, c1_hbm, c2_hbm, zz_hbm,
                  s1_hbm, s2_hbm, zg_hbm,
                  g0, g1, g2, g3, rows1_v, rows2_v, rowsz_v, sem):
    wid = lax.axis_index("s") * 2 + lax.axis_index("c")
    base = wid * CHUNK
    gbufs = (g0, g1, g2, g3)
    for c in range(NCH):
        pltpu.sync_copy(g_hbm.at[pl.ds(base + c * NIDX, NIDX)], gbufs[c])
    cps = []
    for c in range(NCH):
        cps.append(pltpu.async_copy(
            c1_hbm.at[gbufs[c]], rows1_v.at[pl.ds(c * NIDX, NIDX)], sem))
        cps.append(pltpu.async_copy(
            c2_hbm.at[gbufs[c]], rows2_v.at[pl.ds(c * NIDX, NIDX)], sem))
        cps.append(pltpu.async_copy(
            zz_hbm.at[gbufs[c]], rowsz_v.at[pl.ds(c * NIDX, NIDX)], sem))
    for cp in cps:
        cp.wait()
    pltpu.sync_copy(rows1_v, s1_hbm.at[pl.ds(base, CHUNK)])
    pltpu.sync_copy(rows2_v, s2_hbm.at[pl.ds(base, CHUNK)])
    pltpu.sync_copy(rowsz_v, zg_hbm.at[pl.ds(base, CHUNK)])


def _combine_body(dmax_ref, tot_ref, gamma_ref, beta_ref, s_ref,
                  zg_ref, s1_ref, s2_ref, out_ref, sum_s, sq_s):
    p = pl.program_id(0)
    j = pl.program_id(1)

    @pl.when((p == 0) & (j == 0))
    def _init():
        sum_s[...] = jnp.zeros_like(sum_s)
        sq_s[...] = jnp.zeros_like(sq_s)

    sd = s_ref[...] + dmax_ref[...]
    m = jnp.where(sd > 0, sd, 0.2 * sd)
    f1 = jnp.exp(sd - m)
    f2 = jnp.exp(0.2 * sd - m)
    t1v = tot_ref[0:1, :]
    t1s = tot_ref[2:3, 0:1]
    b1 = t1v - s1_ref[...]
    b2 = s2_ref[...]
    y1 = t1s - zg_ref[:, 0:1]
    y2 = zg_ref[:, 1:2]
    den = f1 * y1 + f2 * y2
    o = (f1 * b1 + f2 * b2) / den
    rid = lax.broadcasted_iota(jnp.int32, (T2, 1), 0) + j * T2
    valid = rid < N

    @pl.when(p == 0)
    def _acc():
        om = jnp.where(valid, o, 0.0)
        sum_s[...] += jnp.sum(om, axis=0, keepdims=True)
        sq_s[...] += jnp.sum(om * om, axis=0, keepdims=True)

    @pl.when(p == 1)
    def _wr():
        mean = sum_s[...] * (1.0 / N)
        var = sq_s[...] * (1.0 / N) - mean * mean
        inv = lax.rsqrt(var + 1e-5)
        out_ref[...] = jnp.maximum(
            (o - mean) * inv * gamma_ref[...] + beta_ref[...], 0.0)


_combine = pl.pallas_call(
    _combine_body,
    grid=(2, NT2),
    in_specs=[
        pl.BlockSpec((1, 1), lambda p, j: (0, 0)),
        pl.BlockSpec((8, F), lambda p, j: (0, 0)),
        pl.BlockSpec((1, F), lambda p, j: (0, 0)),
        pl.BlockSpec((1, F), lambda p, j: (0, 0)),
        pl.BlockSpec((T2, 1), lambda p, j: (j, 0)),
        pl.BlockSpec((T2, ZL), lambda p, j: (j, 0)),
        pl.BlockSpec((T2, F), lambda p, j: (j, 0)),
        pl.BlockSpec((T2, F), lambda p, j: (j, 0)),
    ],
    out_specs=pl.BlockSpec((T2, F), lambda p, j: (j, 0)),
    out_shape=jax.ShapeDtypeStruct((NP, F), jnp.float32),
    scratch_shapes=[
        pltpu.VMEM((1, F), jnp.float32),
        pltpu.VMEM((1, F), jnp.float32),
    ],
)


def kernel(x, graph, W, a_src, a_dst, gamma, beta):
    del graph
    h, s, d = _proj(x, W, a_src.reshape(1, F), a_dst.reshape(1, F))
    sortd, perm = lax.sort_key_val(d.reshape(N), jnp.arange(N, dtype=jnp.int32))
    ds_pad = jnp.concatenate([jnp.full((PAD,), NEG, jnp.float32), sortd])
    p_pad = jnp.concatenate([jnp.zeros((PAD,), jnp.int32), perm])
    s_pad = jnp.concatenate([s.reshape(N), jnp.zeros((PAD,), jnp.float32)])
    dmax = sortd[N - 1:N].reshape(1, 1)
    pv = jnp.concatenate(
        [ds_pad[::128], jnp.full((128 - NB,), jnp.inf, jnp.float32)]
    ).reshape(1, 128)
    hp = _make_sc_hgather()(p_pad, h)
    c1, c2, zz, tot, g = _cumsum(
        dmax, ds_pad.reshape(NP, 1), hp, s_pad.reshape(NP, 1),
        ds_pad.reshape(NB, 128), pv)
    s1g, s2g, zg = _make_sc_pick()(g.reshape(NP), c1, c2, zz)
    o = _combine(dmax, tot, gamma.reshape(1, F), beta.reshape(1, F),
                 s_pad.reshape(NP, 1), zg, s1g, s2g)
    return o[:N]


# speedup vs baseline: 2.6841x; 1.1007x over previous
import functools

import jax
import jax.numpy as jnp
from jax import lax
from jax.experimental import pallas as pl
from jax.experimental.pallas import tpu as pltpu
from jax.experimental.pallas import tpu_sc as plsc

N = 10000
F = 128
NW = 32
CHUNK = 320
NP = NW * CHUNK
PAD = NP - N
NEG = -1e30
NIDX = 80
NCH = CHUNK // NIDX
ZL = 128
TILE = 512
NT = NP // TILE
NB = NP // 128
T2 = 512
NT2 = NP // T2


def _proj_body(x_ref, w_ref, asrc_ref, adst_ref, h_ref, s_ref, d_ref):
    h = jnp.dot(x_ref[...], w_ref[...], preferred_element_type=jnp.float32)
    h_ref[...] = h
    s_ref[...] = jnp.sum(h * asrc_ref[...], axis=1, keepdims=True)
    d_ref[...] = jnp.sum(h * adst_ref[...], axis=1, keepdims=True)


_proj = pl.pallas_call(
    _proj_body,
    out_shape=[
        jax.ShapeDtypeStruct((N, F), jnp.float32),
        jax.ShapeDtypeStruct((N, 1), jnp.float32),
        jax.ShapeDtypeStruct((N, 1), jnp.float32),
    ],
)


@functools.cache
def _make_sc_hgather():
    mesh = plsc.VectorSubcoreMesh(core_axis_name="c", subcore_axis_name="s")
    return functools.partial(
        pl.kernel,
        mesh=mesh,
        out_type=jax.ShapeDtypeStruct((NP, F), jnp.float32),
        scratch_types=[
            pltpu.VMEM((NIDX,), jnp.int32),
            pltpu.VMEM((NIDX,), jnp.int32),
            pltpu.VMEM((NIDX,), jnp.int32),
            pltpu.VMEM((NIDX,), jnp.int32),
            pltpu.VMEM((CHUNK, F), jnp.float32),
            pltpu.SemaphoreType.DMA,
        ],
    )(_sc_hgather_body)


def _sc_hgather_body(p_hbm, h_hbm, hp_hbm, p0, p1, p2, p3, rows_v, sem):
    wid = lax.axis_index("s") * 2 + lax.axis_index("c")
    base = wid * CHUNK
    pbufs = (p0, p1, p2, p3)
    for c in range(NCH):
        pltpu.sync_copy(p_hbm.at[pl.ds(base + c * NIDX, NIDX)], pbufs[c])
    cps = []
    for c in range(NCH):
        cps.append(pltpu.async_copy(
            h_hbm.at[pbufs[c]], rows_v.at[pl.ds(c * NIDX, NIDX)], sem))
    for cp in cps:
        cp.wait()
    pltpu.sync_copy(rows_v, hp_hbm.at[pl.ds(base, CHUNK)])


def _cumsum_body(dmax_ref, ds_ref, hp_ref, s_ref, ds3_ref, pv_ref,
                 c1_ref, c2_ref, tot_ref, g_ref, y1_ref, y2_ref,
                 c1c, c2c):
    i = pl.program_id(0)

    @pl.when(i == 0)
    def _init():
        c1c[...] = jnp.zeros_like(c1c)
        c2c[...] = jnp.zeros_like(c2c)

    rid = lax.broadcasted_iota(jnp.int32, (TILE, TILE), 0)
    cid = lax.broadcasted_iota(jnp.int32, (TILE, TILE), 1)
    lt = jnp.where(rid >= cid, 1.0, 0.0).astype(jnp.float32)
    dcol = ds_ref[...]
    dmax = dmax_ref[...]
    w1 = jnp.exp(dcol - dmax)
    w2 = jnp.exp(0.2 * (dcol - dmax))
    tile = hp_ref[...]
    cs1 = jnp.dot(lt, w1 * tile, preferred_element_type=jnp.float32) + c1c[...]
    cs2 = jnp.dot(lt, w2 * tile, preferred_element_type=jnp.float32) + c2c[...]
    c1_ref[...] = cs1
    c2_ref[...] = cs2
    c1c[...] = cs1[TILE - 1:TILE, :]
    c2c[...] = cs2[TILE - 1:TILE, :]

    t = -s_ref[...]
    b = jnp.sum((pv_ref[...] <= t).astype(jnp.float32),
                axis=1, keepdims=True) - 1.0
    cid80 = lax.broadcasted_iota(jnp.int32, (TILE, NB), 1)
    onehot = (cid80 == b.astype(jnp.int32)).astype(jnp.float32)
    brow = jnp.dot(onehot, ds3_ref[...],
                   preferred_element_type=jnp.float32)
    le = brow <= t
    c = jnp.sum(le.astype(jnp.float32), axis=1, keepdims=True)
    g_ref[...] = jnp.maximum(
        b.astype(jnp.int32) * 128 + c.astype(jnp.int32) - 1, 0)

    E1 = jnp.exp(ds3_ref[...] - dmax)
    E2 = jnp.exp(0.2 * (ds3_ref[...] - dmax))
    S1 = jnp.sum(E1, axis=1, keepdims=True)
    S2 = jnp.sum(E2, axis=1, keepdims=True)
    rj = lax.broadcasted_iota(jnp.int32, (NB, NB), 0)
    cj = lax.broadcasted_iota(jnp.int32, (NB, NB), 1)
    pex2 = jnp.dot((rj > cj).astype(jnp.float32), S2,
                   preferred_element_type=jnp.float32)
    qsf1 = jnp.dot((rj < cj).astype(jnp.float32), S1,
                   preferred_element_type=jnp.float32)
    eb1 = jnp.exp(brow - dmax)
    eb2 = jnp.exp(0.2 * (brow - dmax))
    part2 = jnp.sum(jnp.where(le, eb2, 0.0), axis=1, keepdims=True)
    part1 = jnp.sum(jnp.where(le, 0.0, eb1), axis=1, keepdims=True)
    y2_ref[...] = jnp.dot(onehot, pex2,
                          preferred_element_type=jnp.float32) + part2
    y1_ref[...] = jnp.dot(onehot, qsf1,
                          preferred_element_type=jnp.float32) + part1

    @pl.when(i == NT - 1)
    def _fin():
        tot_ref[...] = jnp.concatenate(
            [cs1[TILE - 1:TILE, :], cs2[TILE - 1:TILE, :],
             jnp.zeros((6, F), jnp.float32)], axis=0)


_cumsum = pl.pallas_call(
    _cumsum_body,
    grid=(NT,),
    in_specs=[
        pl.BlockSpec((1, 1), lambda i: (0, 0)),
        pl.BlockSpec((TILE, 1), lambda i: (i, 0)),
        pl.BlockSpec((TILE, F), lambda i: (i, 0)),
        pl.BlockSpec((TILE, 1), lambda i: (i, 0)),
        pl.BlockSpec((NB, 128), lambda i: (0, 0)),
        pl.BlockSpec((1, 128), lambda i: (0, 0)),
    ],
    out_specs=[
        pl.BlockSpec((TILE, F), lambda i: (i, 0)),
        pl.BlockSpec((TILE, F), lambda i: (i, 0)),
        pl.BlockSpec((8, F), lambda i: (0, 0)),
        pl.BlockSpec((TILE, 1), lambda i: (i, 0)),
        pl.BlockSpec((TILE, 1), lambda i: (i, 0)),
        pl.BlockSpec((TILE, 1), lambda i: (i, 0)),
    ],
    out_shape=[
        jax.ShapeDtypeStruct((NP, F), jnp.float32),
        jax.ShapeDtypeStruct((NP, F), jnp.float32),
        jax.ShapeDtypeStruct((8, F), jnp.float32),
        jax.ShapeDtypeStruct((NP, 1), jnp.int32),
        jax.ShapeDtypeStruct((NP, 1), jnp.float32),
        jax.ShapeDtypeStruct((NP, 1), jnp.float32),
    ],
    scratch_shapes=[
        pltpu.VMEM((1, F), jnp.float32),
        pltpu.VMEM((1, F), jnp.float32),
    ],
)


@functools.cache
def _make_sc_pick():
    mesh = plsc.VectorSubcoreMesh(core_axis_name="c", subcore_axis_name="s")
    return functools.partial(
        pl.kernel,
        mesh=mesh,
        out_type=[
            jax.ShapeDtypeStruct((NP, F), jnp.float32),
            jax.ShapeDtypeStruct((NP, F), jnp.float32),
        ],
        scratch_types=[
            pltpu.VMEM((NIDX,), jnp.int32),
            pltpu.VMEM((NIDX,), jnp.int32),
            pltpu.VMEM((NIDX,), jnp.int32),
            pltpu.VMEM((NIDX,), jnp.int32),
            pltpu.VMEM((CHUNK, F), jnp.float32),
            pltpu.VMEM((CHUNK, F), jnp.float32),
            pltpu.SemaphoreType.DMA,
        ],
    )(_sc_pick_body)


def _sc_pick_body(g_hbm, c1_hbm, c2_hbm,
                  s1_hbm, s2_hbm,
                  g0, g1, g2, g3, rows1_v, rows2_v, sem):
    wid = lax.axis_index("s") * 2 + lax.axis_index("c")
    base = wid * CHUNK
    gbufs = (g0, g1, g2, g3)
    for c in range(NCH):
        pltpu.sync_copy(g_hbm.at[pl.ds(base + c * NIDX, NIDX)], gbufs[c])
    cps = []
    for c in range(NCH):
        cps.append(pltpu.async_copy(
            c1_hbm.at[gbufs[c]], rows1_v.at[pl.ds(c * NIDX, NIDX)], sem))
        cps.append(pltpu.async_copy(
            c2_hbm.at[gbufs[c]], rows2_v.at[pl.ds(c * NIDX, NIDX)], sem))
    for cp in cps:
        cp.wait()
    pltpu.sync_copy(rows1_v, s1_hbm.at[pl.ds(base, CHUNK)])
    pltpu.sync_copy(rows2_v, s2_hbm.at[pl.ds(base, CHUNK)])


def _combine_body(dmax_ref, tot_ref, gamma_ref, beta_ref, s_ref,
                  y1_ref, y2_ref, s1_ref, s2_ref, out_ref, sum_s, sq_s):
    p = pl.program_id(0)
    j = pl.program_id(1)

    @pl.when((p == 0) & (j == 0))
    def _init():
        sum_s[...] = jnp.zeros_like(sum_s)
        sq_s[...] = jnp.zeros_like(sq_s)

    sd = s_ref[...] + dmax_ref[...]
    m = jnp.where(sd > 0, sd, 0.2 * sd)
    f1 = jnp.exp(sd - m)
    f2 = jnp.exp(0.2 * sd - m)
    t1v = tot_ref[0:1, :]
    b1 = t1v - s1_ref[...]
    b2 = s2_ref[...]
    den = f1 * y1_ref[...] + f2 * y2_ref[...]
    o = (f1 * b1 + f2 * b2) / den
    rid = lax.broadcasted_iota(jnp.int32, (T2, 1), 0) + j * T2
    valid = rid < N

    @pl.when(p == 0)
    def _acc():
        om = jnp.where(valid, o, 0.0)
        sum_s[...] += jnp.sum(om, axis=0, keepdims=True)
        sq_s[...] += jnp.sum(om * om, axis=0, keepdims=True)

    @pl.when(p == 1)
    def _wr():
        mean = sum_s[...] * (1.0 / N)
        var = sq_s[...] * (1.0 / N) - mean * mean
        inv = lax.rsqrt(var + 1e-5)
        out_ref[...] = jnp.maximum(
            (o - mean) * inv * gamma_ref[...] + beta_ref[...], 0.0)


_combine = pl.pallas_call(
    _combine_body,
    grid=(2, NT2),
    in_specs=[
        pl.BlockSpec((1, 1), lambda p, j: (0, 0)),
        pl.BlockSpec((8, F), lambda p, j: (0, 0)),
        pl.BlockSpec((1, F), lambda p, j: (0, 0)),
        pl.BlockSpec((1, F), lambda p, j: (0, 0)),
        pl.BlockSpec((T2, 1), lambda p, j: (j, 0)),
        pl.BlockSpec((T2, 1), lambda p, j: (j, 0)),
        pl.BlockSpec((T2, 1), lambda p, j: (j, 0)),
        pl.BlockSpec((T2, F), lambda p, j: (j, 0)),
        pl.BlockSpec((T2, F), lambda p, j: (j, 0)),
    ],
    out_specs=pl.BlockSpec((T2, F), lambda p, j: (j, 0)),
    out_shape=jax.ShapeDtypeStruct((NP, F), jnp.float32),
    scratch_shapes=[
        pltpu.VMEM((1, F), jnp.float32),
        pltpu.VMEM((1, F), jnp.float32),
    ],
)


def kernel(x, graph, W, a_src, a_dst, gamma, beta):
    del graph
    h, s, d = _proj(x, W, a_src.reshape(1, F), a_dst.reshape(1, F))
    sortd, perm = lax.sort_key_val(d.reshape(N), jnp.arange(N, dtype=jnp.int32))
    ds_pad = jnp.concatenate([jnp.full((PAD,), NEG, jnp.float32), sortd])
    p_pad = jnp.concatenate([jnp.zeros((PAD,), jnp.int32), perm])
    s_pad = jnp.concatenate([s.reshape(N), jnp.zeros((PAD,), jnp.float32)])
    dmax = sortd[N - 1:N].reshape(1, 1)
    pv = jnp.concatenate(
        [ds_pad[::128], jnp.full((128 - NB,), jnp.inf, jnp.float32)]
    ).reshape(1, 128)
    hp = _make_sc_hgather()(p_pad, h)
    c1, c2, tot, g, y1, y2 = _cumsum(
        dmax, ds_pad.reshape(NP, 1), hp, s_pad.reshape(NP, 1),
        ds_pad.reshape(NB, 128), pv)
    s1g, s2g = _make_sc_pick()(g.reshape(NP), c1, c2)
    o = _combine(dmax, tot, gamma.reshape(1, F), beta.reshape(1, F),
                 s_pad.reshape(NP, 1), y1, y2, s1g, s2g)
    return o[:N]


# speedup vs baseline: 2.7442x; 1.0224x over previous
import functools

import jax
import jax.numpy as jnp
from jax import lax
from jax.experimental import pallas as pl
from jax.experimental.pallas import tpu as pltpu
from jax.experimental.pallas import tpu_sc as plsc

N = 10000
F = 128
NW = 32
CHUNK = 320
NP = NW * CHUNK
PAD = NP - N
NEG = -1e30
NIDX = 80
NCH = CHUNK // NIDX
ZL = 128
TILE = 512
NT = NP // TILE
NB = NP // 128
T2 = 512
NT2 = NP // T2
TR = 2048


def _proj_body(x_ref, w_ref, asrc_ref, adst_ref, h_ref, s_ref, d_ref):
    h = jnp.dot(x_ref[...], w_ref[...], preferred_element_type=jnp.float32)
    h_ref[...] = h
    s_ref[...] = jnp.sum(h * asrc_ref[...], axis=1, keepdims=True)
    d_ref[...] = jnp.sum(h * adst_ref[...], axis=1, keepdims=True)


_proj = pl.pallas_call(
    _proj_body,
    out_shape=[
        jax.ShapeDtypeStruct((N, F), jnp.float32),
        jax.ShapeDtypeStruct((N, 1), jnp.float32),
        jax.ShapeDtypeStruct((N, 1), jnp.float32),
    ],
)


@functools.cache
def _make_sc_hgather():
    mesh = plsc.VectorSubcoreMesh(core_axis_name="c", subcore_axis_name="s")
    return functools.partial(
        pl.kernel,
        mesh=mesh,
        out_type=jax.ShapeDtypeStruct((NP, F), jnp.float32),
        scratch_types=[
            pltpu.VMEM((NIDX,), jnp.int32),
            pltpu.VMEM((NIDX,), jnp.int32),
            pltpu.VMEM((NIDX,), jnp.int32),
            pltpu.VMEM((NIDX,), jnp.int32),
            pltpu.VMEM((CHUNK, F), jnp.float32),
            pltpu.SemaphoreType.DMA,
        ],
    )(_sc_hgather_body)


def _sc_hgather_body(p_hbm, h_hbm, hp_hbm, p0, p1, p2, p3, rows_v, sem):
    wid = lax.axis_index("s") * 2 + lax.axis_index("c")
    base = wid * CHUNK
    pbufs = (p0, p1, p2, p3)
    for c in range(NCH):
        pltpu.sync_copy(p_hbm.at[pl.ds(base + c * NIDX, NIDX)], pbufs[c])
    cps = []
    for c in range(NCH):
        cps.append(pltpu.async_copy(
            h_hbm.at[pbufs[c]], rows_v.at[pl.ds(c * NIDX, NIDX)], sem))
    for cp in cps:
        cp.wait()
    pltpu.sync_copy(rows_v, hp_hbm.at[pl.ds(base, CHUNK)])


def _cumsum_body(dmax_ref, ds_ref, hp_ref,
                 c1_ref, c2_ref, tot_ref,
                 c1c, c2c):
    i = pl.program_id(0)

    @pl.when(i == 0)
    def _init():
        c1c[...] = jnp.zeros_like(c1c)
        c2c[...] = jnp.zeros_like(c2c)

    rid = lax.broadcasted_iota(jnp.int32, (TILE, TILE), 0)
    cid = lax.broadcasted_iota(jnp.int32, (TILE, TILE), 1)
    lt = jnp.where(rid >= cid, 1.0, 0.0).astype(jnp.float32)
    dcol = ds_ref[...]
    dmax = dmax_ref[...]
    w1 = jnp.exp(dcol - dmax)
    w2 = jnp.exp(0.2 * (dcol - dmax))
    tile = hp_ref[...]
    cs1 = jnp.dot(lt, w1 * tile, preferred_element_type=jnp.float32) + c1c[...]
    cs2 = jnp.dot(lt, w2 * tile, preferred_element_type=jnp.float32) + c2c[...]
    c1_ref[...] = cs1
    c2_ref[...] = cs2
    c1c[...] = cs1[TILE - 1:TILE, :]
    c2c[...] = cs2[TILE - 1:TILE, :]

    @pl.when(i == NT - 1)
    def _fin():
        tot_ref[...] = jnp.concatenate(
            [cs1[TILE - 1:TILE, :], cs2[TILE - 1:TILE, :],
             jnp.zeros((6, F), jnp.float32)], axis=0)


_cumsum = pl.pallas_call(
    _cumsum_body,
    grid=(NT,),
    in_specs=[
        pl.BlockSpec((1, 1), lambda i: (0, 0)),
        pl.BlockSpec((TILE, 1), lambda i: (i, 0)),
        pl.BlockSpec((TILE, F), lambda i: (i, 0)),
    ],
    out_specs=[
        pl.BlockSpec((TILE, F), lambda i: (i, 0)),
        pl.BlockSpec((TILE, F), lambda i: (i, 0)),
        pl.BlockSpec((8, F), lambda i: (0, 0)),
    ],
    out_shape=[
        jax.ShapeDtypeStruct((NP, F), jnp.float32),
        jax.ShapeDtypeStruct((NP, F), jnp.float32),
        jax.ShapeDtypeStruct((8, F), jnp.float32),
    ],
    scratch_shapes=[
        pltpu.VMEM((1, F), jnp.float32),
        pltpu.VMEM((1, F), jnp.float32),
    ],
)


def _rank_body(dmax_ref, s_ref, ds3_ref, pv_ref, g_ref, y1_ref, y2_ref):
    dmax = dmax_ref[...]
    t = -s_ref[...]
    b = jnp.sum((pv_ref[...] <= t).astype(jnp.float32),
                axis=1, keepdims=True) - 1.0
    cid80 = lax.broadcasted_iota(jnp.int32, (TR, NB), 1)
    onehot = (cid80 == b.astype(jnp.int32)).astype(jnp.float32)
    brow = jnp.dot(onehot, ds3_ref[...],
                   preferred_element_type=jnp.float32)
    le = brow <= t
    c = jnp.sum(le.astype(jnp.float32), axis=1, keepdims=True)
    g_ref[...] = jnp.maximum(
        b.astype(jnp.int32) * 128 + c.astype(jnp.int32) - 1, 0)

    E1 = jnp.exp(ds3_ref[...] - dmax)
    E2 = jnp.exp(0.2 * (ds3_ref[...] - dmax))
    S1 = jnp.sum(E1, axis=1, keepdims=True)
    S2 = jnp.sum(E2, axis=1, keepdims=True)
    rj = lax.broadcasted_iota(jnp.int32, (NB, NB), 0)
    cj = lax.broadcasted_iota(jnp.int32, (NB, NB), 1)
    pex2 = jnp.dot((rj > cj).astype(jnp.float32), S2,
                   preferred_element_type=jnp.float32)
    qsf1 = jnp.dot((rj < cj).astype(jnp.float32), S1,
                   preferred_element_type=jnp.float32)
    eb1 = jnp.exp(brow - dmax)
    eb2 = jnp.exp(0.2 * (brow - dmax))
    part2 = jnp.sum(jnp.where(le, eb2, 0.0), axis=1, keepdims=True)
    part1 = jnp.sum(jnp.where(le, 0.0, eb1), axis=1, keepdims=True)
    y2_ref[...] = jnp.dot(onehot, pex2,
                          preferred_element_type=jnp.float32) + part2
    y1_ref[...] = jnp.dot(onehot, qsf1,
                          preferred_element_type=jnp.float32) + part1


_rank = pl.pallas_call(
    _rank_body,
    grid=(NP // TR,),
    in_specs=[
        pl.BlockSpec((1, 1), lambda i: (0, 0)),
        pl.BlockSpec((TR, 1), lambda i: (i, 0)),
        pl.BlockSpec((NB, 128), lambda i: (0, 0)),
        pl.BlockSpec((1, 128), lambda i: (0, 0)),
    ],
    out_specs=[
        pl.BlockSpec((TR, 1), lambda i: (i, 0)),
        pl.BlockSpec((TR, 1), lambda i: (i, 0)),
        pl.BlockSpec((TR, 1), lambda i: (i, 0)),
    ],
    out_shape=[
        jax.ShapeDtypeStruct((NP, 1), jnp.int32),
        jax.ShapeDtypeStruct((NP, 1), jnp.float32),
        jax.ShapeDtypeStruct((NP, 1), jnp.float32),
    ],
)


@functools.cache
def _make_sc_pick():
    mesh = plsc.VectorSubcoreMesh(core_axis_name="c", subcore_axis_name="s")
    return functools.partial(
        pl.kernel,
        mesh=mesh,
        out_type=[
            jax.ShapeDtypeStruct((NP, F), jnp.float32),
            jax.ShapeDtypeStruct((NP, F), jnp.float32),
        ],
        scratch_types=[
            pltpu.VMEM((NIDX,), jnp.int32),
            pltpu.VMEM((NIDX,), jnp.int32),
            pltpu.VMEM((NIDX,), jnp.int32),
            pltpu.VMEM((NIDX,), jnp.int32),
            pltpu.VMEM((CHUNK, F), jnp.float32),
            pltpu.VMEM((CHUNK, F), jnp.float32),
            pltpu.SemaphoreType.DMA,
        ],
    )(_sc_pick_body)


def _sc_pick_body(g_hbm, c1_hbm, c2_hbm,
                  s1_hbm, s2_hbm,
                  g0, g1, g2, g3, rows1_v, rows2_v, sem):
    wid = lax.axis_index("s") * 2 + lax.axis_index("c")
    base = wid * CHUNK
    gbufs = (g0, g1, g2, g3)
    for c in range(NCH):
        pltpu.sync_copy(g_hbm.at[pl.ds(base + c * NIDX, NIDX)], gbufs[c])
    cps = []
    for c in range(NCH):
        cps.append(pltpu.async_copy(
            c1_hbm.at[gbufs[c]], rows1_v.at[pl.ds(c * NIDX, NIDX)], sem))
        cps.append(pltpu.async_copy(
            c2_hbm.at[gbufs[c]], rows2_v.at[pl.ds(c * NIDX, NIDX)], sem))
    for cp in cps:
        cp.wait()
    pltpu.sync_copy(rows1_v, s1_hbm.at[pl.ds(base, CHUNK)])
    pltpu.sync_copy(rows2_v, s2_hbm.at[pl.ds(base, CHUNK)])


def _combine_body(dmax_ref, tot_ref, gamma_ref, beta_ref, s_ref,
                  y1_ref, y2_ref, s1_ref, s2_ref, out_ref, sum_s, sq_s):
    p = pl.program_id(0)
    j = pl.program_id(1)

    @pl.when((p == 0) & (j == 0))
    def _init():
        sum_s[...] = jnp.zeros_like(sum_s)
        sq_s[...] = jnp.zeros_like(sq_s)

    sd = s_ref[...] + dmax_ref[...]
    m = jnp.where(sd > 0, sd, 0.2 * sd)
    f1 = jnp.exp(sd - m)
    f2 = jnp.exp(0.2 * sd - m)
    t1v = tot_ref[0:1, :]
    b1 = t1v - s1_ref[...]
    b2 = s2_ref[...]
    den = f1 * y1_ref[...] + f2 * y2_ref[...]
    o = (f1 * b1 + f2 * b2) / den
    rid = lax.broadcasted_iota(jnp.int32, (T2, 1), 0) + j * T2
    valid = rid < N

    @pl.when(p == 0)
    def _acc():
        om = jnp.where(valid, o, 0.0)
        sum_s[...] += jnp.sum(om, axis=0, keepdims=True)
        sq_s[...] += jnp.sum(om * om, axis=0, keepdims=True)

    @pl.when(p == 1)
    def _wr():
        mean = sum_s[...] * (1.0 / N)
        var = sq_s[...] * (1.0 / N) - mean * mean
        inv = lax.rsqrt(var + 1e-5)
        out_ref[...] = jnp.maximum(
            (o - mean) * inv * gamma_ref[...] + beta_ref[...], 0.0)


_combine = pl.pallas_call(
    _combine_body,
    grid=(2, NT2),
    in_specs=[
        pl.BlockSpec((1, 1), lambda p, j: (0, 0)),
        pl.BlockSpec((8, F), lambda p, j: (0, 0)),
        pl.BlockSpec((1, F), lambda p, j: (0, 0)),
        pl.BlockSpec((1, F), lambda p, j: (0, 0)),
        pl.BlockSpec((T2, 1), lambda p, j: (j, 0)),
        pl.BlockSpec((T2, 1), lambda p, j: (j, 0)),
        pl.BlockSpec((T2, 1), lambda p, j: (j, 0)),
        pl.BlockSpec((T2, F), lambda p, j: (j, 0)),
        pl.BlockSpec((T2, F), lambda p, j: (j, 0)),
    ],
    out_specs=pl.BlockSpec((T2, F), lambda p, j: (j, 0)),
    out_shape=jax.ShapeDtypeStruct((NP, F), jnp.float32),
    scratch_shapes=[
        pltpu.VMEM((1, F), jnp.float32),
        pltpu.VMEM((1, F), jnp.float32),
    ],
)


def kernel(x, graph, W, a_src, a_dst, gamma, beta):
    del graph
    h, s, d = _proj(x, W, a_src.reshape(1, F), a_dst.reshape(1, F))
    sortd, perm = lax.sort_key_val(d.reshape(N), jnp.arange(N, dtype=jnp.int32))
    ds_pad = jnp.concatenate([jnp.full((PAD,), NEG, jnp.float32), sortd])
    p_pad = jnp.concatenate([jnp.zeros((PAD,), jnp.int32), perm])
    s_pad = jnp.concatenate([s.reshape(N), jnp.zeros((PAD,), jnp.float32)])
    dmax = sortd[N - 1:N].reshape(1, 1)
    pv = jnp.concatenate(
        [ds_pad[::128], jnp.full((128 - NB,), jnp.inf, jnp.float32)]
    ).reshape(1, 128)
    hp = _make_sc_hgather()(p_pad, h)
    g, y1, y2 = _rank(dmax, s_pad.reshape(NP, 1), ds_pad.reshape(NB, 128), pv)
    c1, c2, tot = _cumsum(dmax, ds_pad.reshape(NP, 1), hp)
    s1g, s2g = _make_sc_pick()(g.reshape(NP), c1, c2)
    o = _combine(dmax, tot, gamma.reshape(1, F), beta.reshape(1, F),
                 s_pad.reshape(NP, 1), y1, y2, s1g, s2g)
    return o[:N]
